# baseline (speedup 1.0000x reference)
import functools

import jax
import jax.numpy as jnp
from jax import lax
from jax.experimental import pallas as pl
from jax.experimental.pallas import tpu as pltpu
from jax.experimental.pallas import tpu_sc as plsc

_B = 1024
_D = 128
_NC = 2
_NS = 16
_NW = _NC * _NS
_BPW = _B // _NW

_mesh = plsc.VectorSubcoreMesh(core_axis_name="c", subcore_axis_name="s")


@functools.partial(
    pl.kernel,
    mesh=_mesh,
    out_type=[
        jax.ShapeDtypeStruct((_B, _D), jnp.float32),
        jax.ShapeDtypeStruct((_B, _D), jnp.float32),
    ],
    scratch_types=[
        pltpu.VMEM((_BPW,), jnp.int32),
        pltpu.VMEM((_BPW, _D), jnp.float32),
        pltpu.VMEM((_BPW,), jnp.int32),
        pltpu.VMEM((_BPW, _D), jnp.float32),
        pltpu.SemaphoreType.DMA,
        pltpu.SemaphoreType.DMA,
    ],
)
def _sc_double_gather(apb_idx_hbm, lsn_idx_hbm, ape_hbm, lsn_hbm,
                      apb_out, lsn_out,
                      idx_a, rows_a, idx_l, rows_l, sem_a, sem_l):
    wid = lax.axis_index("s") * _NC + lax.axis_index("c")
    base = wid * _BPW
    pltpu.sync_copy(apb_idx_hbm.at[pl.ds(base, _BPW)], idx_a)
    pltpu.sync_copy(lsn_idx_hbm.at[pl.ds(base, _BPW)], idx_l)
    cp_a = pltpu.async_copy(ape_hbm.at[idx_a], rows_a, sem_a)
    cp_l = pltpu.async_copy(lsn_hbm.at[idx_l], rows_l, sem_l)
    cp_a.wait()
    cp_l.wait()
    pltpu.sync_copy(rows_a, apb_out.at[pl.ds(base, _BPW)])
    pltpu.sync_copy(rows_l, lsn_out.at[pl.ds(base, _BPW)])


def kernel(spkemb, alsn, tlsn, active_passive_bit, lsn_id, ape_table, lsn_table):
    apb, lsn_rows = _sc_double_gather(
        active_passive_bit.astype(jnp.int32),
        lsn_id.astype(jnp.int32),
        ape_table,
        lsn_table,
    )
    return (spkemb, alsn, tlsn, apb, lsn_rows[:, None, :])
